# baseline (speedup 1.0000x reference)
import functools

import jax
import jax.numpy as jnp
from jax import lax
from jax.experimental import pallas as pl
from jax.experimental.pallas import tpu as pltpu
from jax.experimental.pallas import tpu_sc as plsc

_M, _N, _DIM = 64, 128, 256
_NUM = _M * _N
_ALPHA = 0.3
_SIGMA = max(_M, _N) / 2.0

_NC, _NS, _L = 2, 16, 16
_NW = _NC * _NS
_RPW = _NUM // _NW
_KD = _DIM // _L
_BIG = 2**30

_mesh = plsc.VectorSubcoreMesh(core_axis_name="c", subcore_axis_name="s")
_cparams = pltpu.CompilerParams(needs_layout_passes=False)


def _iota16():
    return lax.broadcasted_iota(jnp.int32, (_L,), 0)


def _wid():
    return lax.axis_index("s") * _NC + lax.axis_index("c")


@functools.partial(
    pl.kernel,
    mesh=_mesh,
    out_type=(
        jax.ShapeDtypeStruct((_NW, _L), jnp.float32),
        jax.ShapeDtypeStruct((_NW, _L), jnp.int32),
    ),
    scratch_types=[
        pltpu.VMEM((_DIM,), jnp.float32),
        pltpu.VMEM((_RPW, _DIM), jnp.float32),
        pltpu.VMEM((_RPW * (_L + 1),), jnp.float32),
        pltpu.VMEM((_L,), jnp.float32),
        pltpu.VMEM((_L,), jnp.int32),
    ],
    compiler_params=_cparams,
)
def _dist_kernel(x_hbm, w_hbm, dist_hbm, idx_hbm, xv, wv, dacc, bd_v, bi_v):
    wid = _wid()
    base = wid * _RPW
    pltpu.sync_copy(x_hbm, xv)
    pltpu.sync_copy(w_hbm.at[pl.ds(base, _RPW)], wv)

    def row_body(r, _):
        acc = jnp.zeros((_L,), jnp.float32)
        for k in range(_KD):
            e = xv[pl.ds(k * _L, _L)] - wv[r, pl.ds(k * _L, _L)]
            acc = acc + e * e
        dacc[pl.ds(r * (_L + 1), _L)] = acc
        return 0

    lax.fori_loop(0, _RPW, row_body, 0, unroll=4)

    lanes = _iota16()
    best_d = jnp.full((_L,), jnp.inf, jnp.float32)
    best_i = jnp.zeros((_L,), jnp.int32)
    for g in range(_RPW // _L):
        rows = jnp.int32(g * _L) + lanes
        s = jnp.zeros((_L,), jnp.float32)
        flat = rows * (_L + 1)
        for k in range(_L):
            s = s + plsc.load_gather(dacc, [flat + k])
        take = s < best_d
        best_d = jnp.where(take, s, best_d)
        best_i = jnp.where(take, base + rows, best_i)

    bd_v[pl.ds(0, _L)] = best_d
    bi_v[pl.ds(0, _L)] = best_i
    pltpu.sync_copy(bd_v, dist_hbm.at[wid])
    pltpu.sync_copy(bi_v, idx_hbm.at[wid])


@functools.partial(
    pl.kernel,
    mesh=_mesh,
    out_type=(
        jax.ShapeDtypeStruct((_L,), jnp.int32),
        jax.ShapeDtypeStruct((_NUM, _DIM), jnp.float32),
    ),
    scratch_types=[
        pltpu.VMEM((_DIM,), jnp.float32),
        pltpu.VMEM((_RPW, _DIM), jnp.float32),
        pltpu.VMEM((_NW, _L), jnp.float32),
        pltpu.VMEM((_NW, _L), jnp.int32),
        pltpu.VMEM((_L,), jnp.int32),
    ],
    compiler_params=_cparams,
)
def _update_kernel(x_hbm, w_hbm, dist_hbm, idx_hbm, misc_hbm, out_hbm,
                   xv, wv, dp, ip, misc_v):
    wid = _wid()
    base = wid * _RPW
    pltpu.sync_copy(x_hbm, xv)
    pltpu.sync_copy(w_hbm.at[pl.ds(base, _RPW)], wv)
    pltpu.sync_copy(dist_hbm, dp)
    pltpu.sync_copy(idx_hbm, ip)

    best_d = dp[0, pl.ds(0, _L)]
    best_i = ip[0, pl.ds(0, _L)]
    for t in range(1, _NW):
        d = dp[t, pl.ds(0, _L)]
        i = ip[t, pl.ds(0, _L)]
        take = d < best_d
        best_d = jnp.where(take, d, best_d)
        best_i = jnp.where(take, i, best_i)
    dmin = jnp.min(best_d)
    bmu = jnp.min(jnp.where(best_d == dmin, best_i, jnp.int32(_BIG)))
    bi = bmu // _N
    bj = bmu - bi * _N
    bif = bi.astype(jnp.float32)
    bjf = bj.astype(jnp.float32)

    lanes = _iota16()

    def group_body(g, _):
        rows = (base + g * _L) + lanes
        rf = rows.astype(jnp.float32)
        rate = 1.0 - rf * jnp.float32(1.0 / _NUM)
        alpha_t = rate * jnp.float32(_ALPHA)
        sigma_t = rate * jnp.float32(_SIGMA)
        ri = rows // _N
        rj = rows - ri * _N
        di = bif - ri.astype(jnp.float32)
        dj = bjf - rj.astype(jnp.float32)
        ld2 = di * di + dj * dj
        h = jnp.exp(-ld2 / (2.0 * sigma_t * sigma_t))
        cvec = h * alpha_t
        for l in range(_L):
            c = cvec[l]
            r = g * _L + l
            for k in range(_KD):
                w16 = wv[r, pl.ds(k * _L, _L)]
                wv[r, pl.ds(k * _L, _L)] = w16 + c * (xv[pl.ds(k * _L, _L)] - w16)
        return 0

    lax.fori_loop(0, _RPW // _L, group_body, 0)
    pltpu.sync_copy(wv, out_hbm.at[pl.ds(base, _RPW)])

    @pl.when(wid == 0)
    def _():
        zero = jnp.zeros((_L,), jnp.int32)
        vec = jnp.where(lanes == 0, bmu, zero)
        vec = jnp.where(lanes == 1, bi, vec)
        vec = jnp.where(lanes == 2, bj, vec)
        misc_v[pl.ds(0, _L)] = vec
        pltpu.sync_copy(misc_v, misc_hbm)


def kernel(x, weights, locations):
    del locations
    dist_part, idx_part = _dist_kernel(x, weights)
    misc, new_w = _update_kernel(x, weights, dist_part, idx_part)
    return misc[0], misc[1:3], new_w


# speedup vs baseline: 1.4931x; 1.4931x over previous
import functools

import jax
import jax.numpy as jnp
from jax import lax
from jax.experimental import pallas as pl
from jax.experimental.pallas import tpu as pltpu
from jax.experimental.pallas import tpu_sc as plsc

_M, _N, _DIM = 64, 128, 256
_NUM = _M * _N
_ALPHA = 0.3
_SIGMA = max(_M, _N) / 2.0

_NC, _NS, _L = 2, 16, 16
_NW = _NC * _NS
_RPW = _NUM // _NW
_KD = _DIM // _L
_NB = 4
_RPB = _RPW // _NB
_GPB = _RPB // _L
_PAD = _L + 1
_BIG = 2**30

_mesh = plsc.VectorSubcoreMesh(core_axis_name="c", subcore_axis_name="s")
_cparams = pltpu.CompilerParams(needs_layout_passes=False)


def _iota16():
    return lax.broadcasted_iota(jnp.int32, (_L,), 0)


def _wid():
    return lax.axis_index("s") * _NC + lax.axis_index("c")


def _tree_sum(vals):
    vals = list(vals)
    while len(vals) > 1:
        vals = [a + b for a, b in zip(vals[::2], vals[1::2])]
    return vals[0]


@functools.partial(
    pl.kernel,
    mesh=_mesh,
    out_type=(
        jax.ShapeDtypeStruct((_NW, _L), jnp.float32),
        jax.ShapeDtypeStruct((_NW, _L), jnp.int32),
    ),
    scratch_types=[
        pltpu.VMEM((_DIM,), jnp.float32),
        pltpu.VMEM((_RPW, _DIM), jnp.float32),
        pltpu.VMEM((_RPW * _PAD,), jnp.float32),
        pltpu.VMEM((_L,), jnp.float32),
        pltpu.VMEM((_L,), jnp.int32),
        pltpu.SemaphoreType.DMA,
        pltpu.SemaphoreType.DMA,
        pltpu.SemaphoreType.DMA,
        pltpu.SemaphoreType.DMA,
        pltpu.SemaphoreType.DMA,
    ],
    compiler_params=_cparams,
)
def _dist_kernel(x_hbm, w_hbm, dist_hbm, idx_hbm,
                 xv, wv, dacc, bd_v, bi_v, sem_x, s0, s1, s2, s3):
    wid = _wid()
    base = wid * _RPW
    sems = [s0, s1, s2, s3]
    hx = pltpu.async_copy(x_hbm, xv, sem_x)
    hw = [
        pltpu.async_copy(
            w_hbm.at[pl.ds(base + b * _RPB, _RPB)],
            wv.at[pl.ds(b * _RPB, _RPB)],
            sems[b],
        )
        for b in range(_NB)
    ]
    hx.wait()
    xs = [xv[pl.ds(k * _L, _L)] for k in range(_KD)]

    for b in range(_NB):
        hw[b].wait()

        def grp_body(g, _, b=b):
            r0 = (b * _GPB + g) * _L
            for l in range(_L):
                r = r0 + l
                es = [xs[k] - wv[r, pl.ds(k * _L, _L)] for k in range(_KD)]
                acc = _tree_sum([e * e for e in es])
                dacc[pl.ds(r * _PAD, _L)] = acc
            return 0

        lax.fori_loop(0, _GPB, grp_body, 0)

    lanes = _iota16()
    best_d = jnp.full((_L,), jnp.inf, jnp.float32)
    best_i = jnp.zeros((_L,), jnp.int32)
    for g in range(_RPW // _L):
        rows = jnp.int32(g * _L) + lanes
        flat = rows * _PAD
        s = _tree_sum([plsc.load_gather(dacc, [flat + k]) for k in range(_L)])
        take = s < best_d
        best_d = jnp.where(take, s, best_d)
        best_i = jnp.where(take, base + rows, best_i)

    bd_v[pl.ds(0, _L)] = best_d
    bi_v[pl.ds(0, _L)] = best_i
    pltpu.sync_copy(bd_v, dist_hbm.at[wid])
    pltpu.sync_copy(bi_v, idx_hbm.at[wid])


@functools.partial(
    pl.kernel,
    mesh=_mesh,
    out_type=(
        jax.ShapeDtypeStruct((_L,), jnp.int32),
        jax.ShapeDtypeStruct((_NUM, _DIM), jnp.float32),
    ),
    scratch_types=[
        pltpu.VMEM((_DIM,), jnp.float32),
        pltpu.VMEM((_RPW, _DIM), jnp.float32),
        pltpu.VMEM((_NW, _L), jnp.float32),
        pltpu.VMEM((_NW, _L), jnp.int32),
        pltpu.VMEM((_L,), jnp.int32),
        pltpu.SemaphoreType.DMA,
        pltpu.SemaphoreType.DMA,
        pltpu.SemaphoreType.DMA,
        pltpu.SemaphoreType.DMA,
        pltpu.SemaphoreType.DMA,
        pltpu.SemaphoreType.DMA,
        pltpu.SemaphoreType.DMA,
    ],
    compiler_params=_cparams,
)
def _update_kernel(x_hbm, w_hbm, dist_hbm, idx_hbm, misc_hbm, out_hbm,
                   xv, wv, dp, ip, misc_v,
                   sem_x, sem_p, s0, s1, s2, s3, sem_o):
    wid = _wid()
    base = wid * _RPW
    sems = [s0, s1, s2, s3]
    hx = pltpu.async_copy(x_hbm, xv, sem_x)
    hd = pltpu.async_copy(dist_hbm, dp, sem_p)
    hi = pltpu.async_copy(idx_hbm, ip, sem_p)
    hw = [
        pltpu.async_copy(
            w_hbm.at[pl.ds(base + b * _RPB, _RPB)],
            wv.at[pl.ds(b * _RPB, _RPB)],
            sems[b],
        )
        for b in range(_NB)
    ]

    hd.wait()
    hi.wait()
    best_d = dp[0, pl.ds(0, _L)]
    best_i = ip[0, pl.ds(0, _L)]
    for t in range(1, _NW):
        d = dp[t, pl.ds(0, _L)]
        i = ip[t, pl.ds(0, _L)]
        take = d < best_d
        best_d = jnp.where(take, d, best_d)
        best_i = jnp.where(take, i, best_i)
    dmin = jnp.min(best_d)
    bmu = jnp.min(jnp.where(best_d == dmin, best_i, jnp.int32(_BIG)))
    bi = bmu // _N
    bj = bmu - bi * _N
    bif = bi.astype(jnp.float32)
    bjf = bj.astype(jnp.float32)

    hx.wait()
    xs = [xv[pl.ds(k * _L, _L)] for k in range(_KD)]
    lanes = _iota16()

    ho = []
    for b in range(_NB):
        hw[b].wait()

        def grp_body(g, _, b=b):
            rows = (base + (b * _GPB + g) * _L) + lanes
            rf = rows.astype(jnp.float32)
            rate = 1.0 - rf * jnp.float32(1.0 / _NUM)
            alpha_t = rate * jnp.float32(_ALPHA)
            sigma_t = rate * jnp.float32(_SIGMA)
            ri = rows // _N
            rj = rows - ri * _N
            di = bif - ri.astype(jnp.float32)
            dj = bjf - rj.astype(jnp.float32)
            ld2 = di * di + dj * dj
            h = jnp.exp(-ld2 / (2.0 * sigma_t * sigma_t))
            cvec = h * alpha_t
            r0 = (b * _GPB + g) * _L
            for l in range(_L):
                c = cvec[l]
                r = r0 + l
                ws = [wv[r, pl.ds(k * _L, _L)] for k in range(_KD)]
                ys = [w + c * (x - w) for w, x in zip(ws, xs)]
                for k in range(_KD):
                    wv[r, pl.ds(k * _L, _L)] = ys[k]
            return 0

        lax.fori_loop(0, _GPB, grp_body, 0)
        ho.append(
            pltpu.async_copy(
                wv.at[pl.ds(b * _RPB, _RPB)],
                out_hbm.at[pl.ds(base + b * _RPB, _RPB)],
                sem_o,
            )
        )

    @pl.when(wid == 0)
    def _():
        zero = jnp.zeros((_L,), jnp.int32)
        vec = jnp.where(lanes == 0, bmu, zero)
        vec = jnp.where(lanes == 1, bi, vec)
        vec = jnp.where(lanes == 2, bj, vec)
        misc_v[pl.ds(0, _L)] = vec
        pltpu.sync_copy(misc_v, misc_hbm)

    for h in ho:
        h.wait()


def kernel(x, weights, locations):
    del locations
    dist_part, idx_part = _dist_kernel(x, weights)
    misc, new_w = _update_kernel(x, weights, dist_part, idx_part)
    return misc[0], misc[1:3], new_w
